# baseline (speedup 1.0000x reference)
import functools

import jax
import jax.numpy as jnp
from jax import lax
from jax.experimental import pallas as pl
from jax.experimental.pallas import tpu as pltpu
from jax.experimental.pallas import tpu_sc as plsc

B = 1024
D = 768
H = 32768
K = 128
DEAD_ICUT = 3906

ENC_HT = 1024
DEC_HT = 512
TK_R = 64

CH = 16
NCH = H // CH
NC, NS = 2, 16
NW = NC * NS
RPW = B // NW
RPG = 16
IDW = 160


def _ln_body(x_ref, bpre_ref, xp_ref, mu_ref, std_ref):
    x = x_ref[...]
    mu = jnp.mean(x, axis=1, keepdims=True)
    xc = x - mu
    var = jnp.sum(xc * xc, axis=1, keepdims=True) * (1.0 / (D - 1))
    std = jnp.sqrt(var)
    xn = xc / (std + 1e-5)
    xp_ref[...] = xn - bpre_ref[...]
    mu_ref[...] = mu
    std_ref[...] = std


def _enc_body(xp_ref, w_ref, benc_ref, out_ref):
    out_ref[...] = (
        jnp.dot(xp_ref[...], w_ref[...], preferred_element_type=jnp.float32)
        + benc_ref[...]
    )


def _kth_thresh(pa):
    b = lax.bitcast_convert_type(pa, jnp.int32)
    s = jnp.where(b < 0, b ^ jnp.int32(0x7FFFFFFF), b)

    def step(it, t):
        bit = 30 - it
        cand = t | (jnp.int32(1) << bit.astype(jnp.int32))
        cnt = jnp.sum(
            jnp.where(s >= cand, 1.0, 0.0), axis=1, keepdims=True
        )
        return jnp.where(cnt >= K, cand, t)

    cnt_pos = jnp.sum(jnp.where(s >= 0, 1.0, 0.0), axis=1, keepdims=True)
    t0 = jnp.where(cnt_pos >= K, jnp.int32(0), jnp.int32(-0x80000000))
    t = lax.fori_loop(0, 31, step, t0)
    tb = jnp.where(t >= 0, t, t ^ jnp.int32(0x7FFFFFFF))
    return lax.bitcast_convert_type(tb, jnp.float32)


def _chunkmax_body(pa_ref, m_ref, tlb_ref):
    pa = pa_ref[...]
    m = jnp.max(pa.reshape(TK_R, NCH, CH), axis=2)
    m_ref[...] = m
    tlb_ref[...] = _kth_thresh(m)


def _topk_small_body(cp_ref, thr_ref):
    thr_ref[...] = _kth_thresh(cp_ref[...])


def _topk_full_body(pa_ref, thr_ref):
    thr_ref[...] = _kth_thresh(pa_ref[...])


def _sc_compact_body(m_hbm, tlb_hbm, pre2_hbm, cp_hbm, ov_hbm,
                     m_v, t_v, idx_v, g_v, ov_v, sem):
    wid = lax.axis_index("s") * NC + lax.axis_index("c")
    row0 = wid * RPW
    pltpu.sync_copy(tlb_hbm.at[pl.ds(row0, RPW)], t_v)
    ovv = jnp.zeros((16,), jnp.int32)
    iota = lax.iota(jnp.int32, 16)

    for g in range(RPW // RPG):
        gr0 = row0 + g * RPG
        pltpu.sync_copy(m_hbm.at[pl.ds(gr0, RPG)], m_v)
        tg = t_v[pl.ds(g * RPG, 16)]
        for r in range(RPG):
            thrv = jnp.full((16,), tg[r])
            rowbase = (gr0 + r) * NCH

            def chunk_step(c, off, r=r, thrv=thrv, rowbase=rowbase):
                mv = m_v[r, pl.ds(c * CH, 16)]
                msk = mv >= thrv
                ids = rowbase + c * CH + iota

                @pl.when(off < K)
                def _():
                    plsc.store_compressed(
                        idx_v.at[pl.ds(r * IDW + off, 16)], ids, mask=msk)

                pc = plsc.all_reduce_population_count(msk)
                return off + pc[0]

            total = lax.fori_loop(0, NCH // 16, chunk_step,
                                  jnp.int32(0))
            ovv = ovv | jnp.where(
                jnp.full((16,), total) > K, jnp.int32(1), jnp.int32(0))

        copies = []
        for r in range(RPG):
            copies.append(pltpu.async_copy(
                pre2_hbm.at[idx_v.at[pl.ds(r * IDW, K)]], g_v.at[r], sem))
        for c in copies:
            c.wait()
        pltpu.sync_copy(g_v, cp_hbm.at[pl.ds(gr0, RPG)])

    idx_v[pl.ds(0, 16)] = ovv
    pltpu.sync_copy(idx_v.at[pl.ds(0, 16)],
                    ov_hbm.at[pl.ds(wid * 16, 16)])


def _dec_body(pa_ref, wd_ref, stats_ref, thr_ref, bpre_ref, mu_ref,
              std_ref, lat_ref, ns_ref, nd_ref, out_ref, acc_ref, nd_acc):
    i = pl.program_id(0)
    pa = pa_ref[...]
    lat = jnp.where(pa >= thr_ref[...], jnp.maximum(pa, 0.0), 0.0)
    lat_ref[...] = lat

    colcnt = jnp.sum(jnp.where(lat > 0.0, 1.0, 0.0), axis=0, keepdims=True)
    ns = stats_ref[...] * (colcnt == 0.0).astype(jnp.int32) + 1
    ns_ref[...] = ns

    @pl.when(i == 0)
    def _():
        acc_ref[...] = jnp.zeros_like(acc_ref)
        nd_acc[0] = 0

    nd_acc[0] += jnp.sum((ns > DEAD_ICUT).astype(jnp.int32))
    acc_ref[...] += jnp.dot(lat, wd_ref[...],
                            preferred_element_type=jnp.float32)

    @pl.when(i == pl.num_programs(0) - 1)
    def _():
        out_ref[...] = (acc_ref[...] + bpre_ref[...]) * std_ref[...] \
            + mu_ref[...]
        nd_ref[0, 0] = nd_acc[0]


def kernel(x, w_enc, w_dec, b_enc, b_pre, stats_last_nonzero):
    f32 = jnp.float32
    bpre2 = b_pre.reshape(1, D)
    benc2 = b_enc.reshape(1, H)
    stats2 = stats_last_nonzero.reshape(1, H)

    xp, mu, std = pl.pallas_call(
        _ln_body,
        out_shape=[
            jax.ShapeDtypeStruct((B, D), f32),
            jax.ShapeDtypeStruct((B, 1), f32),
            jax.ShapeDtypeStruct((B, 1), f32),
        ],
    )(x, bpre2)

    pre = pl.pallas_call(
        _enc_body,
        grid=(H // ENC_HT,),
        in_specs=[
            pl.BlockSpec((B, D), lambda i: (0, 0)),
            pl.BlockSpec((D, ENC_HT), lambda i: (0, i)),
            pl.BlockSpec((1, ENC_HT), lambda i: (0, i)),
        ],
        out_specs=pl.BlockSpec((B, ENC_HT), lambda i: (0, i)),
        out_shape=jax.ShapeDtypeStruct((B, H), f32),
    )(xp, w_enc, benc2)

    thr = pl.pallas_call(
        _topk_full_body,
        grid=(B // TK_R,),
        in_specs=[pl.BlockSpec((TK_R, H), lambda i: (i, 0))],
        out_specs=pl.BlockSpec((TK_R, 1), lambda i: (i, 0)),
        out_shape=jax.ShapeDtypeStruct((B, 1), f32),
    )(pre)

    lat, ns2, nd, rec = pl.pallas_call(
        _dec_body,
        grid=(H // DEC_HT,),
        in_specs=[
            pl.BlockSpec((B, DEC_HT), lambda i: (0, i)),
            pl.BlockSpec((DEC_HT, D), lambda i: (i, 0)),
            pl.BlockSpec((1, DEC_HT), lambda i: (0, i)),
            pl.BlockSpec((B, 1), lambda i: (0, 0)),
            pl.BlockSpec((1, D), lambda i: (0, 0)),
            pl.BlockSpec((B, 1), lambda i: (0, 0)),
            pl.BlockSpec((B, 1), lambda i: (0, 0)),
        ],
        out_specs=[
            pl.BlockSpec((B, DEC_HT), lambda i: (0, i)),
            pl.BlockSpec((1, DEC_HT), lambda i: (0, i)),
            pl.BlockSpec(memory_space=pltpu.SMEM),
            pl.BlockSpec((B, D), lambda i: (0, 0)),
        ],
        out_shape=[
            jax.ShapeDtypeStruct((B, H), f32),
            jax.ShapeDtypeStruct((1, H), jnp.int32),
            jax.ShapeDtypeStruct((1, 1), jnp.int32),
            jax.ShapeDtypeStruct((B, D), f32),
        ],
        scratch_shapes=[
            pltpu.VMEM((B, D), f32),
            pltpu.SMEM((1,), jnp.int32),
        ],
    )(pre, w_dec, stats2, thr, bpre2, mu, std)

    return (rec, nd[0, 0], lat, ns2.reshape(H))


# speedup vs baseline: 1.0255x; 1.0255x over previous
import functools

import jax
import jax.numpy as jnp
from jax import lax
from jax.experimental import pallas as pl
from jax.experimental.pallas import tpu as pltpu
from jax.experimental.pallas import tpu_sc as plsc

B = 1024
D = 768
H = 32768
K = 128
DEAD_ICUT = 3906

ENC_HT = 1024
DEC_HT = 1024
TK_R = 64

CH = 16
NCH = H // CH
NC, NS = 2, 16
NW = NC * NS
RPW = B // NW
RPG = 16
IDW = 160


def _ln_body(x_ref, bpre_ref, xp_ref, mu_ref, std_ref):
    x = x_ref[...]
    mu = jnp.mean(x, axis=1, keepdims=True)
    xc = x - mu
    var = jnp.sum(xc * xc, axis=1, keepdims=True) * (1.0 / (D - 1))
    std = jnp.sqrt(var)
    xn = xc / (std + 1e-5)
    xp_ref[...] = xn - bpre_ref[...]
    mu_ref[...] = mu
    std_ref[...] = std


def _enc_body(xp_ref, w_ref, benc_ref, out_ref):
    out_ref[...] = (
        jnp.dot(xp_ref[...], w_ref[...], preferred_element_type=jnp.float32)
        + benc_ref[...]
    )


def _kth_thresh(pa):
    b = lax.bitcast_convert_type(pa, jnp.int32)
    s = jnp.where(b < 0, b ^ jnp.int32(0x7FFFFFFF), b)

    def step(it, t):
        bit = 30 - it
        cand = t | (jnp.int32(1) << bit.astype(jnp.int32))
        cnt = jnp.sum(
            jnp.where(s >= cand, 1.0, 0.0), axis=1, keepdims=True
        )
        return jnp.where(cnt >= K, cand, t)

    cnt_pos = jnp.sum(jnp.where(s >= 0, 1.0, 0.0), axis=1, keepdims=True)
    t0 = jnp.where(cnt_pos >= K, jnp.int32(0), jnp.int32(-0x80000000))
    t = lax.fori_loop(0, 31, step, t0)
    tb = jnp.where(t >= 0, t, t ^ jnp.int32(0x7FFFFFFF))
    return lax.bitcast_convert_type(tb, jnp.float32)


def _chunkmax_body(pa_ref, m_ref, tlb_ref):
    pa = pa_ref[...]
    m = jnp.max(pa.reshape(TK_R, NCH, CH), axis=2)
    m_ref[...] = m
    tlb_ref[...] = _kth_thresh(m)


def _topk_small_body(cp_ref, thr_ref):
    thr_ref[...] = _kth_thresh(cp_ref[...])


def _topk_full_body(pa_ref, thr_ref):
    thr_ref[...] = _kth_thresh(pa_ref[...])


def _sc_compact_body(m_hbm, tlb_hbm, pre2_hbm, cp_hbm, ov_hbm,
                     m_v, t_v, idx_v, g_v, ov_v, sem):
    wid = lax.axis_index("s") * NC + lax.axis_index("c")
    row0 = wid * RPW
    pltpu.sync_copy(tlb_hbm.at[pl.ds(row0, RPW)], t_v)
    ovv = jnp.zeros((16,), jnp.int32)
    iota = lax.iota(jnp.int32, 16)

    for g in range(RPW // RPG):
        gr0 = row0 + g * RPG
        pltpu.sync_copy(m_hbm.at[pl.ds(gr0, RPG)], m_v)
        tg = t_v[pl.ds(g * RPG, 16)]
        for r in range(RPG):
            thrv = jnp.full((16,), tg[r])
            rowbase = (gr0 + r) * NCH

            def chunk_step(c, off, r=r, thrv=thrv, rowbase=rowbase):
                mv = m_v[r, pl.ds(c * CH, 16)]
                msk = mv >= thrv
                ids = rowbase + c * CH + iota

                @pl.when(off < K)
                def _():
                    plsc.store_compressed(
                        idx_v.at[pl.ds(r * IDW + off, 16)], ids, mask=msk)

                pc = plsc.all_reduce_population_count(msk)
                return off + pc[0]

            total = lax.fori_loop(0, NCH // 16, chunk_step,
                                  jnp.int32(0))
            ovv = ovv | jnp.where(
                jnp.full((16,), total) > K, jnp.int32(1), jnp.int32(0))

        copies = []
        for r in range(RPG):
            copies.append(pltpu.async_copy(
                pre2_hbm.at[idx_v.at[pl.ds(r * IDW, K)]], g_v.at[r], sem))
        for c in copies:
            c.wait()
        pltpu.sync_copy(g_v, cp_hbm.at[pl.ds(gr0, RPG)])

    idx_v[pl.ds(0, 16)] = ovv
    pltpu.sync_copy(idx_v.at[pl.ds(0, 16)],
                    ov_hbm.at[pl.ds(wid * 16, 16)])


def _dec_body(pa_ref, wd_ref, stats_ref, thr_ref, bpre_ref, mu_ref,
              std_ref, lat_ref, ns_ref, nd_ref, out_ref, acc_ref, nd_acc):
    i = pl.program_id(0)
    pa = pa_ref[...]
    lat = jnp.where(pa >= thr_ref[...], jnp.maximum(pa, 0.0), 0.0)
    lat_ref[...] = lat

    colcnt = jnp.sum(jnp.where(lat > 0.0, 1.0, 0.0), axis=0, keepdims=True)
    ns = stats_ref[...] * (colcnt == 0.0).astype(jnp.int32) + 1
    ns_ref[...] = ns

    @pl.when(i == 0)
    def _():
        acc_ref[...] = jnp.zeros_like(acc_ref)
        nd_acc[0] = 0

    nd_acc[0] += jnp.sum((ns > DEAD_ICUT).astype(jnp.int32))
    acc_ref[...] += jnp.dot(lat, wd_ref[...],
                            preferred_element_type=jnp.float32)

    @pl.when(i == pl.num_programs(0) - 1)
    def _():
        out_ref[...] = (acc_ref[...] + bpre_ref[...]) * std_ref[...] \
            + mu_ref[...]
        nd_ref[0, 0] = nd_acc[0]


def kernel(x, w_enc, w_dec, b_enc, b_pre, stats_last_nonzero):
    f32 = jnp.float32
    bpre2 = b_pre.reshape(1, D)
    benc2 = b_enc.reshape(1, H)
    stats2 = stats_last_nonzero.reshape(1, H)

    xp, mu, std = pl.pallas_call(
        _ln_body,
        out_shape=[
            jax.ShapeDtypeStruct((B, D), f32),
            jax.ShapeDtypeStruct((B, 1), f32),
            jax.ShapeDtypeStruct((B, 1), f32),
        ],
    )(x, bpre2)

    pre = pl.pallas_call(
        _enc_body,
        grid=(H // ENC_HT,),
        in_specs=[
            pl.BlockSpec((B, D), lambda i: (0, 0)),
            pl.BlockSpec((D, ENC_HT), lambda i: (0, i)),
            pl.BlockSpec((1, ENC_HT), lambda i: (0, i)),
        ],
        out_specs=pl.BlockSpec((B, ENC_HT), lambda i: (0, i)),
        out_shape=jax.ShapeDtypeStruct((B, H), f32),
    )(xp, w_enc, benc2)

    thr = pl.pallas_call(
        _topk_full_body,
        grid=(B // TK_R,),
        in_specs=[pl.BlockSpec((TK_R, H), lambda i: (i, 0))],
        out_specs=pl.BlockSpec((TK_R, 1), lambda i: (i, 0)),
        out_shape=jax.ShapeDtypeStruct((B, 1), f32),
    )(pre)

    lat, ns2, nd, rec = pl.pallas_call(
        _dec_body,
        grid=(H // DEC_HT,),
        in_specs=[
            pl.BlockSpec((B, DEC_HT), lambda i: (0, i)),
            pl.BlockSpec((DEC_HT, D), lambda i: (i, 0)),
            pl.BlockSpec((1, DEC_HT), lambda i: (0, i)),
            pl.BlockSpec((B, 1), lambda i: (0, 0)),
            pl.BlockSpec((1, D), lambda i: (0, 0)),
            pl.BlockSpec((B, 1), lambda i: (0, 0)),
            pl.BlockSpec((B, 1), lambda i: (0, 0)),
        ],
        out_specs=[
            pl.BlockSpec((B, DEC_HT), lambda i: (0, i)),
            pl.BlockSpec((1, DEC_HT), lambda i: (0, i)),
            pl.BlockSpec(memory_space=pltpu.SMEM),
            pl.BlockSpec((B, D), lambda i: (0, 0)),
        ],
        out_shape=[
            jax.ShapeDtypeStruct((B, H), f32),
            jax.ShapeDtypeStruct((1, H), jnp.int32),
            jax.ShapeDtypeStruct((1, 1), jnp.int32),
            jax.ShapeDtypeStruct((B, D), f32),
        ],
        scratch_shapes=[
            pltpu.VMEM((B, D), f32),
            pltpu.SMEM((1,), jnp.int32),
        ],
    )(pre, w_dec, stats2, thr, bpre2, mu, std)

    return (rec, nd[0, 0], lat, ns2.reshape(H))


# speedup vs baseline: 1.0846x; 1.0576x over previous
import functools

import jax
import jax.numpy as jnp
from jax import lax
from jax.experimental import pallas as pl
from jax.experimental.pallas import tpu as pltpu
from jax.experimental.pallas import tpu_sc as plsc

B = 1024
D = 768
H = 32768
K = 128
DEAD_ICUT = 3906

ENC_HT = 2048
DEC_HT = 1024
TK_R = 128

CH = 16
NCH = H // CH
NC, NS = 2, 16
NW = NC * NS
RPW = B // NW
RPG = 16
IDW = 160


def _ln_body(x_ref, bpre_ref, xp_ref, mu_ref, std_ref):
    x = x_ref[...]
    mu = jnp.mean(x, axis=1, keepdims=True)
    xc = x - mu
    var = jnp.sum(xc * xc, axis=1, keepdims=True) * (1.0 / (D - 1))
    std = jnp.sqrt(var)
    xn = xc / (std + 1e-5)
    xp_ref[...] = xn - bpre_ref[...]
    mu_ref[...] = mu
    std_ref[...] = std


def _enc_body(xp_ref, w_ref, benc_ref, out_ref):
    out_ref[...] = (
        jnp.dot(xp_ref[...], w_ref[...], preferred_element_type=jnp.float32)
        + benc_ref[...]
    )


def _kth_thresh(pa):
    b = lax.bitcast_convert_type(pa, jnp.int32)
    s = jnp.where(b < 0, b ^ jnp.int32(0x7FFFFFFF), b)

    def step(it, t):
        bit = 30 - it
        cand = t | (jnp.int32(1) << bit.astype(jnp.int32))
        cnt = jnp.sum(
            jnp.where(s >= cand, 1.0, 0.0), axis=1, keepdims=True
        )
        return jnp.where(cnt >= K, cand, t)

    cnt_pos = jnp.sum(jnp.where(s >= 0, 1.0, 0.0), axis=1, keepdims=True)
    t0 = jnp.where(cnt_pos >= K, jnp.int32(0), jnp.int32(-0x80000000))
    t = lax.fori_loop(0, 31, step, t0)
    tb = jnp.where(t >= 0, t, t ^ jnp.int32(0x7FFFFFFF))
    return lax.bitcast_convert_type(tb, jnp.float32)


def _chunkmax_body(pa_ref, m_ref, tlb_ref):
    pa = pa_ref[...]
    m = jnp.max(pa.reshape(TK_R, NCH, CH), axis=2)
    m_ref[...] = m
    tlb_ref[...] = _kth_thresh(m)


def _topk_small_body(cp_ref, thr_ref):
    thr_ref[...] = _kth_thresh(cp_ref[...])


def _topk_full_body(pa_ref, thr_ref):
    thr_ref[...] = _kth_thresh(pa_ref[...])


def _sc_compact_body(m_hbm, tlb_hbm, pre2_hbm, cp_hbm, ov_hbm,
                     m_v, t_v, idx_v, g_v, ov_v, sem):
    wid = lax.axis_index("s") * NC + lax.axis_index("c")
    row0 = wid * RPW
    pltpu.sync_copy(tlb_hbm.at[pl.ds(row0, RPW)], t_v)
    ovv = jnp.zeros((16,), jnp.int32)
    iota = lax.iota(jnp.int32, 16)

    for g in range(RPW // RPG):
        gr0 = row0 + g * RPG
        pltpu.sync_copy(m_hbm.at[pl.ds(gr0, RPG)], m_v)
        tg = t_v[pl.ds(g * RPG, 16)]
        for r in range(RPG):
            thrv = jnp.full((16,), tg[r])
            rowbase = (gr0 + r) * NCH

            def chunk_step(c, off, r=r, thrv=thrv, rowbase=rowbase):
                mv = m_v[r, pl.ds(c * CH, 16)]
                msk = mv >= thrv
                ids = rowbase + c * CH + iota

                @pl.when(off < K)
                def _():
                    plsc.store_compressed(
                        idx_v.at[pl.ds(r * IDW + off, 16)], ids, mask=msk)

                pc = plsc.all_reduce_population_count(msk)
                return off + pc[0]

            total = lax.fori_loop(0, NCH // 16, chunk_step,
                                  jnp.int32(0))
            ovv = ovv | jnp.where(
                jnp.full((16,), total) > K, jnp.int32(1), jnp.int32(0))

        copies = []
        for r in range(RPG):
            copies.append(pltpu.async_copy(
                pre2_hbm.at[idx_v.at[pl.ds(r * IDW, K)]], g_v.at[r], sem))
        for c in copies:
            c.wait()
        pltpu.sync_copy(g_v, cp_hbm.at[pl.ds(gr0, RPG)])

    idx_v[pl.ds(0, 16)] = ovv
    pltpu.sync_copy(idx_v.at[pl.ds(0, 16)],
                    ov_hbm.at[pl.ds(wid * 16, 16)])


def _dec_body(pa_ref, wd_ref, stats_ref, thr_ref, bpre_ref, mu_ref,
              std_ref, lat_ref, ns_ref, nd_ref, out_ref, acc_ref, nd_acc):
    i = pl.program_id(0)
    pa = pa_ref[...]
    lat = jnp.where(pa >= thr_ref[...], jnp.maximum(pa, 0.0), 0.0)
    lat_ref[...] = lat

    colcnt = jnp.sum(jnp.where(lat > 0.0, 1.0, 0.0), axis=0, keepdims=True)
    ns = stats_ref[...] * (colcnt == 0.0).astype(jnp.int32) + 1
    ns_ref[...] = ns

    @pl.when(i == 0)
    def _():
        acc_ref[...] = jnp.zeros_like(acc_ref)
        nd_acc[0] = 0

    nd_acc[0] += jnp.sum((ns > DEAD_ICUT).astype(jnp.int32))
    acc_ref[...] += jnp.dot(lat, wd_ref[...],
                            preferred_element_type=jnp.float32)

    @pl.when(i == pl.num_programs(0) - 1)
    def _():
        out_ref[...] = (acc_ref[...] + bpre_ref[...]) * std_ref[...] \
            + mu_ref[...]
        nd_ref[0, 0] = nd_acc[0]


def kernel(x, w_enc, w_dec, b_enc, b_pre, stats_last_nonzero):
    f32 = jnp.float32
    bpre2 = b_pre.reshape(1, D)
    benc2 = b_enc.reshape(1, H)
    stats2 = stats_last_nonzero.reshape(1, H)

    xp, mu, std = pl.pallas_call(
        _ln_body,
        out_shape=[
            jax.ShapeDtypeStruct((B, D), f32),
            jax.ShapeDtypeStruct((B, 1), f32),
            jax.ShapeDtypeStruct((B, 1), f32),
        ],
    )(x, bpre2)

    pre = pl.pallas_call(
        _enc_body,
        grid=(H // ENC_HT,),
        in_specs=[
            pl.BlockSpec((B, D), lambda i: (0, 0)),
            pl.BlockSpec((D, ENC_HT), lambda i: (0, i)),
            pl.BlockSpec((1, ENC_HT), lambda i: (0, i)),
        ],
        out_specs=pl.BlockSpec((B, ENC_HT), lambda i: (0, i)),
        out_shape=jax.ShapeDtypeStruct((B, H), f32),
    )(xp, w_enc, benc2)

    thr = pl.pallas_call(
        _topk_full_body,
        grid=(B // TK_R,),
        in_specs=[pl.BlockSpec((TK_R, H), lambda i: (i, 0))],
        out_specs=pl.BlockSpec((TK_R, 1), lambda i: (i, 0)),
        out_shape=jax.ShapeDtypeStruct((B, 1), f32),
    )(pre)

    lat, ns2, nd, rec = pl.pallas_call(
        _dec_body,
        grid=(H // DEC_HT,),
        in_specs=[
            pl.BlockSpec((B, DEC_HT), lambda i: (0, i)),
            pl.BlockSpec((DEC_HT, D), lambda i: (i, 0)),
            pl.BlockSpec((1, DEC_HT), lambda i: (0, i)),
            pl.BlockSpec((B, 1), lambda i: (0, 0)),
            pl.BlockSpec((1, D), lambda i: (0, 0)),
            pl.BlockSpec((B, 1), lambda i: (0, 0)),
            pl.BlockSpec((B, 1), lambda i: (0, 0)),
        ],
        out_specs=[
            pl.BlockSpec((B, DEC_HT), lambda i: (0, i)),
            pl.BlockSpec((1, DEC_HT), lambda i: (0, i)),
            pl.BlockSpec(memory_space=pltpu.SMEM),
            pl.BlockSpec((B, D), lambda i: (0, 0)),
        ],
        out_shape=[
            jax.ShapeDtypeStruct((B, H), f32),
            jax.ShapeDtypeStruct((1, H), jnp.int32),
            jax.ShapeDtypeStruct((1, 1), jnp.int32),
            jax.ShapeDtypeStruct((B, D), f32),
        ],
        scratch_shapes=[
            pltpu.VMEM((B, D), f32),
            pltpu.SMEM((1,), jnp.int32),
        ],
    )(pre, w_dec, stats2, thr, bpre2, mu, std)

    return (rec, nd[0, 0], lat, ns2.reshape(H))
